# Initial kernel scaffold; baseline (speedup 1.0000x reference)
import jax
import jax.numpy as jnp
from jax.experimental import pallas as pl


def kernel(logits, labels):
    raise NotImplementedError("write your pallas kernel here")


# speedup vs baseline: 24.1651x; 24.1651x over previous
import functools
import math

import jax
import jax.numpy as jnp
import numpy as np
from jax import lax
from jax.experimental import pallas as pl
from jax.experimental.pallas import tpu as pltpu

_THRESH = 0.7
_N_MIN = 131072
_CT_F32 = np.float32(-math.log(_THRESH))
_CT_KEY = int(np.array(_CT_F32, np.float32).view(np.int32))
_KEY_HI = 0x7F800000


def _nll_body(lg_ref, lab_ref, nll_ref):
    lg = lg_ref[0]
    lab = lab_ref[0]
    c = lg.shape[0]
    m = jnp.max(lg, axis=0)
    s = jnp.sum(jnp.exp(lg - m[None]), axis=0)
    cidx = lax.broadcasted_iota(jnp.int32, lg.shape, 0)
    x = jnp.sum(jnp.where(cidx == lab[None], lg, 0.0), axis=0)
    nll_ref[0] = m + jnp.log(s) - x


def _select_body(n_pix, nll_ref, out_ref):
    nll = nll_ref[...]
    keys = lax.bitcast_convert_type(nll, jnp.int32)
    target = jnp.int32(n_pix - _N_MIN)

    def step(_, carry):
        lo, hi = carry
        mid = lo + (hi - lo) // 2
        cnt = jnp.sum((keys <= mid).astype(jnp.int32))
        return (jnp.where(cnt >= target, lo, mid + 1),
                jnp.where(cnt >= target, mid, hi))

    lo, hi = lax.fori_loop(0, 31, step, (jnp.int32(0), jnp.int32(_KEY_HI)))
    t_key = jnp.minimum(lo, jnp.int32(_CT_KEY))
    valid = keys >= t_key
    cnt_v = jnp.sum(valid.astype(jnp.int32))
    s = jnp.sum(jnp.where(valid, nll, 0.0))
    out_ref[0, 0] = s / jnp.maximum(cnt_v.astype(jnp.float32), 1.0)


def kernel(logits, labels):
    n, c, h, w = logits.shape
    bh = 64
    nll = pl.pallas_call(
        _nll_body,
        grid=(n, h // bh),
        in_specs=[
            pl.BlockSpec((1, c, bh, w), lambda i, j: (i, 0, j, 0)),
            pl.BlockSpec((1, bh, w), lambda i, j: (i, j, 0)),
        ],
        out_specs=pl.BlockSpec((1, bh, w), lambda i, j: (i, j, 0)),
        out_shape=jax.ShapeDtypeStruct((n, h, w), jnp.float32),
    )(logits, labels)

    n_pix = n * h * w
    nll2 = nll.reshape(n_pix // 512, 512)
    loss = pl.pallas_call(
        functools.partial(_select_body, n_pix),
        out_specs=pl.BlockSpec(memory_space=pltpu.SMEM),
        out_shape=jax.ShapeDtypeStruct((1, 1), jnp.float32),
    )(nll2)
    return loss[0, 0]


# speedup vs baseline: 38.9004x; 1.6098x over previous
import functools
import math

import jax
import jax.numpy as jnp
import numpy as np
from jax import lax
from jax.experimental import pallas as pl
from jax.experimental.pallas import tpu as pltpu

_THRESH = 0.7
_N_MIN = 131072
_CT_F32 = np.float32(-math.log(_THRESH))
_CT_KEY = int(np.array(_CT_F32, np.float32).view(np.int32))
_KEY_HI = 0x7F800000


def _nll_body(lg_ref, lab_ref, nll_ref, c0_ref, s0_ref):
    lg = lg_ref[0]
    lab = lab_ref[0]
    m = jnp.max(lg, axis=0)
    s = jnp.sum(jnp.exp(lg - m[None]), axis=0)
    cidx = lax.broadcasted_iota(jnp.int32, lg.shape, 0)
    x = jnp.sum(jnp.where(cidx == lab[None], lg, 0.0), axis=0)
    nll = m + jnp.log(s) - x
    nll_ref[0] = nll
    mask = nll >= _CT_F32
    c0_p = jnp.sum(mask.astype(jnp.float32))
    s0_p = jnp.sum(jnp.where(mask, nll, 0.0))
    first = (pl.program_id(0) == 0) & (pl.program_id(1) == 0)

    @pl.when(first)
    def _init():
        c0_ref[0, 0] = c0_p
        s0_ref[0, 0] = s0_p

    @pl.when(jnp.logical_not(first))
    def _acc():
        c0_ref[0, 0] += c0_p
        s0_ref[0, 0] += s0_p


def _select_body(n_pix, nll_ref, out_ref):
    nll = nll_ref[...]
    keys = lax.bitcast_convert_type(nll, jnp.int32)
    target = jnp.int32(n_pix - _N_MIN)

    def step(_, carry):
        lo, hi = carry
        mid = lo + (hi - lo) // 2
        cnt = jnp.sum((keys <= mid).astype(jnp.int32))
        return (jnp.where(cnt >= target, lo, mid + 1),
                jnp.where(cnt >= target, mid, hi))

    lo, hi = lax.fori_loop(0, 31, step, (jnp.int32(0), jnp.int32(_KEY_HI)))
    t_key = jnp.minimum(lo, jnp.int32(_CT_KEY))
    valid = keys >= t_key
    cnt_v = jnp.sum(valid.astype(jnp.int32))
    s = jnp.sum(jnp.where(valid, nll, 0.0))
    out_ref[0, 0] = s / jnp.maximum(cnt_v.astype(jnp.float32), 1.0)


def kernel(logits, labels):
    n, c, h, w = logits.shape
    bh = 64
    nll, c0, s0 = pl.pallas_call(
        _nll_body,
        grid=(n, h // bh),
        in_specs=[
            pl.BlockSpec((1, c, bh, w), lambda i, j: (i, 0, j, 0)),
            pl.BlockSpec((1, bh, w), lambda i, j: (i, j, 0)),
        ],
        out_specs=[
            pl.BlockSpec((1, bh, w), lambda i, j: (i, j, 0)),
            pl.BlockSpec((1, 1), lambda i, j: (0, 0), memory_space=pltpu.SMEM),
            pl.BlockSpec((1, 1), lambda i, j: (0, 0), memory_space=pltpu.SMEM),
        ],
        out_shape=[
            jax.ShapeDtypeStruct((n, h, w), jnp.float32),
            jax.ShapeDtypeStruct((1, 1), jnp.float32),
            jax.ShapeDtypeStruct((1, 1), jnp.float32),
        ],
    )(logits, labels)

    n_pix = n * h * w
    c0v = c0[0, 0]
    s0v = s0[0, 0]

    def _slow(_):
        nll2 = nll.reshape(n_pix // 512, 512)
        loss = pl.pallas_call(
            functools.partial(_select_body, n_pix),
            out_specs=pl.BlockSpec(memory_space=pltpu.SMEM),
            out_shape=jax.ShapeDtypeStruct((1, 1), jnp.float32),
        )(nll2)
        return loss[0, 0]

    return lax.cond(c0v > _N_MIN, lambda _: s0v / c0v, _slow, operand=None)
